# baseline (speedup 1.0000x reference)
import numpy as np
import jax
import jax.numpy as jnp
from jax.experimental import pallas as pl
from jax.experimental.pallas import tpu as pltpu

N = 5000
B = 128

_NEG = float(np.finfo(np.float32).min)
_BF = jnp.bfloat16


def _toeplitz(wt, win, wout):
    kh, kw, ci, co = wt.shape
    sel = np.stack([np.eye(win, dtype=np.float32)[dx:dx + wout, :]
                    for dx in range(kw)])
    t = jnp.einsum('dox,edcf->excof', sel, wt)
    return t.reshape(kh, win * ci, wout * co)


def _parity(w, wout, co):
    idx = np.concatenate([np.arange(0, wout, 2), np.arange(1, wout, 2)])
    perm = (idx[:, None] * co + np.arange(co)[None, :]).reshape(-1)
    return w[..., perm]


def _pairs(wt):
    kh, kw, ci, co = wt.shape
    flat = wt.reshape(kh, kw * ci, co)
    z = jnp.zeros((kh, ci, co), jnp.float32)
    col0 = jnp.concatenate([flat, z], axis=1)
    col1 = jnp.concatenate([z, flat], axis=1)
    return jnp.concatenate([col0, col1], axis=2)


def _act(acc, b, a):
    y = (acc + b).astype(_BF)
    return jnp.where(y >= 0, y, a * y)


def _bdot(a, b):
    return jnp.dot(a, b, preferred_element_type=jnp.float32)


def _fdot(a, b):
    return jnp.dot(a, b, preferred_element_type=jnp.float32)


def _neg(shape):
    return jnp.full(shape, _NEG, _BF)


def _onet_block(x_ref, w1_ref, w2p_ref, w2l_ref, w3p_ref, w4_ref, w5_ref,
                w6_ref, b1_ref, a1_ref, b2_ref, a2_ref, b3_ref, a3_ref,
                b4_ref, a4_ref, b5_ref, a5_ref, b6_ref, out_ref):
    X = x_ref[...]

    patch = jnp.concatenate(
        [X[dy:dy + 46, :, 48 * ci:48 * ci + 48]
         for ci in range(3) for dy in range(3)], axis=2)
    pe = patch.reshape(46 * B, 432)
    w1 = w1_ref[...]
    ye = _act(_bdot(pe, w1[:, 0:736]), b1_ref[:, 0:736],
              a1_ref[:, 0:736]).reshape(46, B, 736)
    yo = _act(_bdot(pe, w1[:, 736:1472]), b1_ref[:, 736:1472],
              a1_ref[:, 736:1472]).reshape(46, B, 736)
    es = jnp.concatenate([ye[:, :, 32:736], _neg((46, B, 32))], axis=2)
    m = jnp.maximum(jnp.maximum(ye, yo), es)
    m = jnp.concatenate([m, _neg((2, B, 736))], axis=0).reshape(24, 2, B, 736)
    e, o = m[:, 0], m[:, 1]
    p = jnp.maximum(jnp.maximum(e[0:23], o[0:23]), e[1:24])

    pp = jnp.concatenate([p, _neg((23, B, 32))], axis=2)
    e = pp.reshape(23, B, 6, 128)[:, :, 0:5]
    o = pp[:, :, 64:704].reshape(23, B, 5, 128)
    patch = jnp.stack([e, o], axis=3).reshape(23, B, 10, 128)
    acc = _bdot(patch[0:21].reshape(B * 210, 128), w2p_ref[0])
    accl = _bdot(p[0:21, :, 640:736].reshape(B * 21, 96), w2l_ref[0])
    for dy in range(1, 3):
        acc = acc + _bdot(patch[dy:dy + 21].reshape(B * 210, 128),
                          w2p_ref[dy])
        accl = accl + _bdot(p[dy:dy + 21, :, 640:736].reshape(B * 21, 96),
                            w2l_ref[dy])
    y = _act(acc, b2_ref[...], a2_ref[...]).reshape(21, B, 10, 128)
    yl = _act(accl, b2_ref[:, 0:64], a2_ref[:, 0:64]).reshape(21, B, 1, 64)
    e, o = y[..., 0:64], y[..., 64:128]
    en = jnp.concatenate([e[:, :, 1:10], yl], axis=2)
    m = jnp.maximum(jnp.maximum(e, o), en)
    m = jnp.concatenate([m, _neg((1, B, 10, 64))],
                        axis=0).reshape(11, 2, B, 10, 64)
    e, o = m[:, 0], m[:, 1]
    p = jnp.maximum(jnp.maximum(e[0:10], o[0:10]), e[1:11])
    p = p.reshape(10, B, 640)

    patch = jnp.concatenate([p[:, :, 0:512].reshape(10, B, 4, 128),
                             p[:, :, 128:640].reshape(10, B, 4, 128)],
                            axis=3)
    acc = _bdot(patch[0:8].reshape(B * 32, 256), w3p_ref[0])
    for dy in range(1, 3):
        acc = acc + _bdot(patch[dy:dy + 8].reshape(B * 32, 256),
                          w3p_ref[dy])
    y = _act(acc, b3_ref[...], a3_ref[...]).reshape(8, B, 4, 128)
    y = jnp.maximum(y[..., 0:64], y[..., 64:128]).reshape(4, 2, B, 4, 64)
    p = jnp.maximum(y[:, 0], y[:, 1]).reshape(4, B, 256)

    acc = _bdot(p[0:3].reshape(B * 3, 256), w4_ref[0])
    acc = acc + _bdot(p[1:4].reshape(B * 3, 256), w4_ref[1])
    y = _act(acc, b4_ref[...], a4_ref[...]).reshape(3, B, 384)

    acc = _fdot(y[0], w5_ref[0])
    acc = acc + _fdot(y[1], w5_ref[1]) + _fdot(y[2], w5_ref[2])
    y = _act(acc, b5_ref[...], a5_ref[...])
    z = _fdot(y, w6_ref[...]) + b6_ref[...]
    l = z[:, 14:16]
    mx = jnp.max(l, axis=1, keepdims=True)
    ex = jnp.exp(l - mx)
    probs = ex / jnp.sum(ex, axis=1, keepdims=True)
    out_ref[...] = jnp.concatenate([z[:, 0:14], probs], axis=1)


def kernel(x, conv1_w, conv1_b, prelu1_a, conv2_w, conv2_b, prelu2_a,
           conv3_w, conv3_b, prelu3_a, conv4_w, conv4_b, prelu4_a,
           fc5_w, fc5_b, prelu5_a, fc61_w, fc61_b, fc62_w, fc62_b,
           fc63_w, fc63_b):
    n = x.shape[0]
    m = -(-n // B) * B
    bf = lambda a: a.astype(_BF)
    xt = bf(jnp.transpose(x, (2, 0, 1, 3)).reshape(48, n, 144))
    if m != n:
        xt = jnp.pad(xt, ((0, 0), (0, m - n), (0, 0)))

    w1t = jnp.transpose(conv1_w, (2, 3, 1, 0))
    w1 = bf(_parity(jnp.concatenate(
        [_toeplitz(w1t[:, :, ci:ci + 1, :], 48, 46)[dy]
         for ci in range(3) for dy in range(3)], axis=0), 46, 32))
    w2t = jnp.transpose(conv2_w, (2, 3, 1, 0))
    w2p = bf(_pairs(w2t))
    w2l = bf(w2t.reshape(3, 96, 64))
    w3t = jnp.transpose(conv3_w, (2, 3, 1, 0))
    w3p = bf(_pairs(w3t))
    w4 = bf(_toeplitz(jnp.transpose(conv4_w, (2, 3, 1, 0)), 4, 3))
    w5 = bf(jnp.transpose(fc5_w.reshape(256, 128, 3, 3),
                          (3, 2, 1, 0)).reshape(3, 384, 256))
    w6 = bf(jnp.concatenate([fc63_w, fc62_w, fc61_w], axis=0).T)
    b6 = jnp.concatenate([fc63_b, fc62_b, fc61_b], axis=0)

    tile = lambda v, k: jnp.tile(v, k).reshape(1, -1)
    btile = lambda v, k: bf(jnp.tile(v, k).reshape(1, -1))
    full = lambda a: pl.BlockSpec(a.shape, lambda i: (0,) * a.ndim)
    weights = [w1, w2p, w2l, w3p, w4, w5, w6,
               btile(conv1_b, 46), btile(prelu1_a, 46),
               btile(conv2_b, 2), btile(prelu2_a, 2),
               btile(conv3_b, 2), btile(prelu3_a, 2),
               btile(conv4_b, 3), btile(prelu4_a, 3),
               fc5_b.reshape(1, -1), bf(prelu5_a.reshape(1, -1)),
               b6.reshape(1, -1)]

    out = pl.pallas_call(
        _onet_block,
        grid=(m // B,),
        in_specs=[pl.BlockSpec((48, B, 144), lambda i: (0, i, 0))]
                 + [full(a) for a in weights],
        out_specs=pl.BlockSpec((B, 16), lambda i: (i, 0)),
        out_shape=jax.ShapeDtypeStruct((m, 16), jnp.float32),
        compiler_params=pltpu.CompilerParams(
            dimension_semantics=("parallel",)),
    )(xt, *weights)

    return out[:n, 0:10], out[:n, 10:14], out[:n, 14:16]


# speedup vs baseline: 1.1913x; 1.1913x over previous
import numpy as np
import jax
import jax.numpy as jnp
from jax.experimental import pallas as pl
from jax.experimental.pallas import tpu as pltpu

N = 5000
B = 40

_NEG = float(np.finfo(np.float32).min)
_BF = jnp.bfloat16


def _toeplitz(wt, win, wout):
    kh, kw, ci, co = wt.shape
    sel = np.stack([np.eye(win, dtype=np.float32)[dx:dx + wout, :]
                    for dx in range(kw)])
    t = jnp.einsum('dox,edcf->excof', sel, wt)
    return t.reshape(kh, win * ci, wout * co)


def _parity(w, wout, co):
    idx = np.concatenate([np.arange(0, wout, 2), np.arange(1, wout, 2)])
    perm = (idx[:, None] * co + np.arange(co)[None, :]).reshape(-1)
    return w[..., perm]


def _pairs(wt):
    kh, kw, ci, co = wt.shape
    flat = wt.reshape(kh, kw * ci, co)
    z = jnp.zeros((kh, ci, co), jnp.float32)
    col0 = jnp.concatenate([flat, z], axis=1)
    col1 = jnp.concatenate([z, flat], axis=1)
    return jnp.concatenate([col0, col1], axis=2)


def _act(acc, b, a):
    y = (acc + b).astype(_BF)
    return jnp.where(y >= 0, y, a * y)


def _bdot(a, b):
    return jnp.dot(a, b, preferred_element_type=jnp.float32)


def _fdot(a, b):
    return jnp.dot(a, b, preferred_element_type=jnp.float32)


def _neg(shape):
    return jnp.full(shape, _NEG, _BF)


def _onet_block(x_ref, w1_ref, w2p_ref, w2l_ref, w3p_ref, w4_ref, w5_ref,
                w6_ref, b1_ref, a1_ref, b2_ref, a2_ref, b3_ref, a3_ref,
                b4_ref, a4_ref, b5_ref, a5_ref, b6_ref, out_ref):
    X = x_ref[...]

    patch = jnp.concatenate(
        [X[dy:dy + 46, :, 48 * ci:48 * ci + 48]
         for ci in range(3) for dy in range(3)], axis=2)
    pe = patch.reshape(46 * B, 432)
    w1 = w1_ref[...]
    ye = _act(_bdot(pe, w1[:, 0:736]), b1_ref[:, 0:736],
              a1_ref[:, 0:736]).reshape(46, B, 736)
    yo = _act(_bdot(pe, w1[:, 736:1472]), b1_ref[:, 736:1472],
              a1_ref[:, 736:1472]).reshape(46, B, 736)
    es = jnp.concatenate([ye[:, :, 32:736], _neg((46, B, 32))], axis=2)
    m = jnp.maximum(jnp.maximum(ye, yo), es)
    m = jnp.concatenate([m, _neg((2, B, 736))], axis=0).reshape(24, 2, B, 736)
    e, o = m[:, 0], m[:, 1]
    p = jnp.maximum(jnp.maximum(e[0:23], o[0:23]), e[1:24])

    pp = jnp.concatenate([p, _neg((23, B, 32))], axis=2)
    e = pp.reshape(23, B, 6, 128)[:, :, 0:5]
    o = pp[:, :, 64:704].reshape(23, B, 5, 128)
    patch = jnp.stack([e, o], axis=3).reshape(23, B, 10, 128)
    acc = _bdot(patch[0:21].reshape(B * 210, 128), w2p_ref[0])
    accl = _bdot(p[0:21, :, 640:736].reshape(B * 21, 96), w2l_ref[0])
    for dy in range(1, 3):
        acc = acc + _bdot(patch[dy:dy + 21].reshape(B * 210, 128),
                          w2p_ref[dy])
        accl = accl + _bdot(p[dy:dy + 21, :, 640:736].reshape(B * 21, 96),
                            w2l_ref[dy])
    y = _act(acc, b2_ref[...], a2_ref[...]).reshape(21, B, 10, 128)
    yl = _act(accl, b2_ref[:, 0:64], a2_ref[:, 0:64]).reshape(21, B, 1, 64)
    e, o = y[..., 0:64], y[..., 64:128]
    en = jnp.concatenate([e[:, :, 1:10], yl], axis=2)
    m = jnp.maximum(jnp.maximum(e, o), en)
    m = jnp.concatenate([m, _neg((1, B, 10, 64))],
                        axis=0).reshape(11, 2, B, 10, 64)
    e, o = m[:, 0], m[:, 1]
    p = jnp.maximum(jnp.maximum(e[0:10], o[0:10]), e[1:11])
    p = p.reshape(10, B, 640)

    patch = jnp.concatenate([p[:, :, 0:512].reshape(10, B, 4, 128),
                             p[:, :, 128:640].reshape(10, B, 4, 128)],
                            axis=3)
    acc = _bdot(patch[0:8].reshape(B * 32, 256), w3p_ref[0])
    for dy in range(1, 3):
        acc = acc + _bdot(patch[dy:dy + 8].reshape(B * 32, 256),
                          w3p_ref[dy])
    y = _act(acc, b3_ref[...], a3_ref[...]).reshape(8, B, 4, 128)
    y = jnp.maximum(y[..., 0:64], y[..., 64:128]).reshape(4, 2, B, 4, 64)
    p = jnp.maximum(y[:, 0], y[:, 1]).reshape(4, B, 256)

    acc = _bdot(p[0:3].reshape(B * 3, 256), w4_ref[0])
    acc = acc + _bdot(p[1:4].reshape(B * 3, 256), w4_ref[1])
    y = _act(acc, b4_ref[...], a4_ref[...]).reshape(3, B, 384)

    acc = _fdot(y[0], w5_ref[0])
    acc = acc + _fdot(y[1], w5_ref[1]) + _fdot(y[2], w5_ref[2])
    y = _act(acc, b5_ref[...], a5_ref[...])
    z = _fdot(y, w6_ref[...]) + b6_ref[...]
    l = z[:, 14:16]
    mx = jnp.max(l, axis=1, keepdims=True)
    ex = jnp.exp(l - mx)
    probs = ex / jnp.sum(ex, axis=1, keepdims=True)
    out_ref[...] = jnp.concatenate([z[:, 0:14], probs], axis=1)


def kernel(x, conv1_w, conv1_b, prelu1_a, conv2_w, conv2_b, prelu2_a,
           conv3_w, conv3_b, prelu3_a, conv4_w, conv4_b, prelu4_a,
           fc5_w, fc5_b, prelu5_a, fc61_w, fc61_b, fc62_w, fc62_b,
           fc63_w, fc63_b):
    n = x.shape[0]
    m = -(-n // B) * B
    bf = lambda a: a.astype(_BF)
    xt = bf(jnp.transpose(x, (2, 0, 1, 3)).reshape(48, n, 144))
    if m != n:
        xt = jnp.pad(xt, ((0, 0), (0, m - n), (0, 0)))

    w1t = jnp.transpose(conv1_w, (2, 3, 1, 0))
    w1 = bf(_parity(jnp.concatenate(
        [_toeplitz(w1t[:, :, ci:ci + 1, :], 48, 46)[dy]
         for ci in range(3) for dy in range(3)], axis=0), 46, 32))
    w2t = jnp.transpose(conv2_w, (2, 3, 1, 0))
    w2p = bf(_pairs(w2t))
    w2l = bf(w2t.reshape(3, 96, 64))
    w3t = jnp.transpose(conv3_w, (2, 3, 1, 0))
    w3p = bf(_pairs(w3t))
    w4 = bf(_toeplitz(jnp.transpose(conv4_w, (2, 3, 1, 0)), 4, 3))
    w5 = bf(jnp.transpose(fc5_w.reshape(256, 128, 3, 3),
                          (3, 2, 1, 0)).reshape(3, 384, 256))
    w6 = bf(jnp.concatenate([fc63_w, fc62_w, fc61_w], axis=0).T)
    b6 = jnp.concatenate([fc63_b, fc62_b, fc61_b], axis=0)

    tile = lambda v, k: jnp.tile(v, k).reshape(1, -1)
    btile = lambda v, k: bf(jnp.tile(v, k).reshape(1, -1))
    full = lambda a: pl.BlockSpec(a.shape, lambda i: (0,) * a.ndim)
    weights = [w1, w2p, w2l, w3p, w4, w5, w6,
               btile(conv1_b, 46), btile(prelu1_a, 46),
               btile(conv2_b, 2), btile(prelu2_a, 2),
               btile(conv3_b, 2), btile(prelu3_a, 2),
               btile(conv4_b, 3), btile(prelu4_a, 3),
               fc5_b.reshape(1, -1), bf(prelu5_a.reshape(1, -1)),
               b6.reshape(1, -1)]

    out = pl.pallas_call(
        _onet_block,
        grid=(m // B,),
        in_specs=[pl.BlockSpec((48, B, 144), lambda i: (0, i, 0))]
                 + [full(a) for a in weights],
        out_specs=pl.BlockSpec((B, 16), lambda i: (i, 0)),
        out_shape=jax.ShapeDtypeStruct((m, 16), jnp.float32),
        compiler_params=pltpu.CompilerParams(
            dimension_semantics=("parallel",)),
    )(xt, *weights)

    return out[:n, 0:10], out[:n, 10:14], out[:n, 14:16]


# speedup vs baseline: 1.2192x; 1.0235x over previous
import numpy as np
import jax
import jax.numpy as jnp
from jax.experimental import pallas as pl
from jax.experimental.pallas import tpu as pltpu

N = 5000
B = 40

_NEG = float(np.finfo(np.float32).min)
_BF = jnp.bfloat16


def _toeplitz(wt, win, wout):
    kh, kw, ci, co = wt.shape
    sel = np.stack([np.eye(win, dtype=np.float32)[dx:dx + wout, :]
                    for dx in range(kw)])
    t = jnp.einsum('dox,edcf->excof', sel, wt)
    return t.reshape(kh, win * ci, wout * co)


def _parity(w, wout, co):
    idx = np.concatenate([np.arange(0, wout, 2), np.arange(1, wout, 2)])
    perm = (idx[:, None] * co + np.arange(co)[None, :]).reshape(-1)
    return w[..., perm]


def _pairs(wt):
    kh, kw, ci, co = wt.shape
    flat = wt.reshape(kh, kw * ci, co)
    z = jnp.zeros((kh, ci, co), jnp.float32)
    col0 = jnp.concatenate([flat, z], axis=1)
    col1 = jnp.concatenate([z, flat], axis=1)
    return jnp.concatenate([col0, col1], axis=2)


def _act(acc, b, a):
    y = (acc + b).astype(_BF)
    return jnp.where(y >= 0, y, a * y)


def _bdot(a, b):
    return jnp.dot(a, b, preferred_element_type=jnp.float32)


def _fdot(a, b):
    return jnp.dot(a, b, preferred_element_type=jnp.float32)


def _neg(shape):
    return jnp.full(shape, _NEG, _BF)


def _onet_block(x_ref, w1_ref, w2p_ref, w2l_ref, w3p_ref, w4_ref, w5_ref,
                w6_ref, b1_ref, a1_ref, b2_ref, a2_ref, b3_ref, a3_ref,
                b4_ref, a4_ref, b5_ref, a5_ref, b6_ref, out_ref):
    X = x_ref[...]

    patch = jnp.concatenate(
        [X[dy:dy + 46, :, 48 * ci:48 * ci + 48]
         for ci in range(3) for dy in range(3)], axis=2)
    y = _act(_bdot(patch.reshape(46 * B, 432), w1_ref[...]),
             b1_ref[...], a1_ref[...]).reshape(46, B, 1472)
    ye, yo = y[:, :, 0:736], y[:, :, 736:1472]
    es = jnp.concatenate([ye[:, :, 32:736], _neg((46, B, 32))], axis=2)
    m = jnp.maximum(jnp.maximum(ye, yo), es)
    m = jnp.concatenate([m, _neg((2, B, 736))], axis=0).reshape(24, 2, B, 736)
    e, o = m[:, 0], m[:, 1]
    p = jnp.maximum(jnp.maximum(e[0:23], o[0:23]), e[1:24])

    pp = jnp.concatenate([p, _neg((23, B, 32))], axis=2)
    e = pp.reshape(23, B, 6, 128)[:, :, 0:5]
    o = pp[:, :, 64:704].reshape(23, B, 5, 128)
    patch = jnp.stack([e, o], axis=3).reshape(23, B, 10, 128)
    acc = _bdot(patch[0:21].reshape(B * 210, 128), w2p_ref[0])
    accl = _bdot(p[0:21, :, 640:736].reshape(B * 21, 96), w2l_ref[0])
    for dy in range(1, 3):
        acc = acc + _bdot(patch[dy:dy + 21].reshape(B * 210, 128),
                          w2p_ref[dy])
        accl = accl + _bdot(p[dy:dy + 21, :, 640:736].reshape(B * 21, 96),
                            w2l_ref[dy])
    y = _act(acc, b2_ref[...], a2_ref[...]).reshape(21, B, 10, 128)
    yl = _act(accl, b2_ref[:, 0:64], a2_ref[:, 0:64]).reshape(21, B, 1, 64)
    e, o = y[..., 0:64], y[..., 64:128]
    en = jnp.concatenate([e[:, :, 1:10], yl], axis=2)
    m = jnp.maximum(jnp.maximum(e, o), en)
    m = jnp.concatenate([m, _neg((1, B, 10, 64))],
                        axis=0).reshape(11, 2, B, 10, 64)
    e, o = m[:, 0], m[:, 1]
    p = jnp.maximum(jnp.maximum(e[0:10], o[0:10]), e[1:11])
    p = p.reshape(10, B, 640)

    patch = jnp.concatenate([p[:, :, 0:512].reshape(10, B, 4, 128),
                             p[:, :, 128:640].reshape(10, B, 4, 128)],
                            axis=3)
    acc = _bdot(patch[0:8].reshape(B * 32, 256), w3p_ref[0])
    for dy in range(1, 3):
        acc = acc + _bdot(patch[dy:dy + 8].reshape(B * 32, 256),
                          w3p_ref[dy])
    y = _act(acc, b3_ref[...], a3_ref[...]).reshape(8, B, 4, 128)
    y = jnp.maximum(y[..., 0:64], y[..., 64:128]).reshape(4, 2, B, 4, 64)
    p = jnp.maximum(y[:, 0], y[:, 1]).reshape(4, B, 256)

    acc = _bdot(p[0:3].reshape(B * 3, 256), w4_ref[0])
    acc = acc + _bdot(p[1:4].reshape(B * 3, 256), w4_ref[1])
    y = _act(acc, b4_ref[...], a4_ref[...]).reshape(3, B, 384)

    acc = _fdot(y[0], w5_ref[0])
    acc = acc + _fdot(y[1], w5_ref[1]) + _fdot(y[2], w5_ref[2])
    y = _act(acc, b5_ref[...], a5_ref[...])
    z = _fdot(y, w6_ref[...]) + b6_ref[...]
    l = z[:, 14:16]
    mx = jnp.max(l, axis=1, keepdims=True)
    ex = jnp.exp(l - mx)
    probs = ex / jnp.sum(ex, axis=1, keepdims=True)
    out_ref[...] = jnp.concatenate([z[:, 0:14], probs], axis=1)


def kernel(x, conv1_w, conv1_b, prelu1_a, conv2_w, conv2_b, prelu2_a,
           conv3_w, conv3_b, prelu3_a, conv4_w, conv4_b, prelu4_a,
           fc5_w, fc5_b, prelu5_a, fc61_w, fc61_b, fc62_w, fc62_b,
           fc63_w, fc63_b):
    n = x.shape[0]
    m = -(-n // B) * B
    bf = lambda a: a.astype(_BF)
    xt = bf(jnp.transpose(x, (2, 0, 1, 3)).reshape(48, n, 144))
    if m != n:
        xt = jnp.pad(xt, ((0, 0), (0, m - n), (0, 0)))

    w1t = jnp.transpose(conv1_w, (2, 3, 1, 0))
    w1 = bf(_parity(jnp.concatenate(
        [_toeplitz(w1t[:, :, ci:ci + 1, :], 48, 46)[dy]
         for ci in range(3) for dy in range(3)], axis=0), 46, 32))
    w2t = jnp.transpose(conv2_w, (2, 3, 1, 0))
    w2p = bf(_pairs(w2t))
    w2l = bf(w2t.reshape(3, 96, 64))
    w3t = jnp.transpose(conv3_w, (2, 3, 1, 0))
    w3p = bf(_pairs(w3t))
    w4 = bf(_toeplitz(jnp.transpose(conv4_w, (2, 3, 1, 0)), 4, 3))
    w5 = bf(jnp.transpose(fc5_w.reshape(256, 128, 3, 3),
                          (3, 2, 1, 0)).reshape(3, 384, 256))
    w6 = bf(jnp.concatenate([fc63_w, fc62_w, fc61_w], axis=0).T)
    b6 = jnp.concatenate([fc63_b, fc62_b, fc61_b], axis=0)

    tile = lambda v, k: jnp.tile(v, k).reshape(1, -1)
    btile = lambda v, k: bf(jnp.tile(v, k).reshape(1, -1))
    full = lambda a: pl.BlockSpec(a.shape, lambda i: (0,) * a.ndim)
    weights = [w1, w2p, w2l, w3p, w4, w5, w6,
               btile(conv1_b, 46), btile(prelu1_a, 46),
               btile(conv2_b, 2), btile(prelu2_a, 2),
               btile(conv3_b, 2), btile(prelu3_a, 2),
               btile(conv4_b, 3), btile(prelu4_a, 3),
               fc5_b.reshape(1, -1), bf(prelu5_a.reshape(1, -1)),
               b6.reshape(1, -1)]

    out = pl.pallas_call(
        _onet_block,
        grid=(m // B,),
        in_specs=[pl.BlockSpec((48, B, 144), lambda i: (0, i, 0))]
                 + [full(a) for a in weights],
        out_specs=pl.BlockSpec((B, 16), lambda i: (i, 0)),
        out_shape=jax.ShapeDtypeStruct((m, 16), jnp.float32),
        compiler_params=pltpu.CompilerParams(
            dimension_semantics=("parallel",)),
    )(xt, *weights)

    return out[:n, 0:10], out[:n, 10:14], out[:n, 14:16]
